# baseline (speedup 1.0000x reference)
import functools

import jax
import jax.numpy as jnp
from jax import lax
from jax.experimental import pallas as pl
from jax.experimental.pallas import tpu as pltpu
from jax.experimental.pallas import tpu_sc as plsc

B = 16384
D = 32
NR = 200
NC = 2
NS = 16
NW = NC * NS
BPW = B // NW
CB = 128
CH = BPW // CB
G16 = CB // 16


def _sc_body(nodes_hbm, rels_hbm, maps_hbm, ent4_hbm, relf_hbm, out_hbm,
             nodes_v, rels_v, typed_v, gid_v, relf_v, egrp_v, out_v, s1, s2):
    c = lax.axis_index("c")
    s = lax.axis_index("s")
    wid = s * NC + c
    lanes = lax.iota(jnp.int32, 16)

    pltpu.sync_copy(nodes_hbm.at[wid], nodes_v)
    l1 = [pltpu.async_copy(maps_hbm.at[nodes_v.at[j]], typed_v.at[j], s1)
          for j in range(CH)]
    pltpu.sync_copy(rels_hbm.at[wid], rels_v)
    pltpu.sync_copy(relf_hbm, relf_v)

    l2 = []
    for j in range(CH):
        l1[j].wait()

        def gid_row(g, _, j=j):
            sl = pl.ds(g * 16, 16)
            gid_v.at[j][sl] = lax.shift_right_logical(typed_v.at[j][sl], 2)
            return 0

        lax.fori_loop(0, G16, gid_row, 0)
        l2.append(pltpu.async_copy(ent4_hbm.at[gid_v.at[j]], egrp_v.at[j], s2))

    for j in range(CH):
        l2[j].wait()

        def extract(g, _, j=j):
            sl = pl.ds(g * 16, 16)
            tv = typed_v.at[j][sl]
            rv = rels_v.at[j][sl]
            esub = (tv & 3) * D
            rbase = rv * D
            erow = g * 16 + lanes
            oi = j * CB + g * 16 + lanes
            orow = lax.shift_right_logical(oi, 1)
            ocol = (oi & 1) * (2 * D)
            for cc in range(D):
                ev = plsc.load_gather(egrp_v.at[j], [erow, esub + cc])
                plsc.store_scatter(out_v, [orow, ocol + cc], ev)
                rvv = plsc.load_gather(relf_v, [rbase + cc])
                plsc.store_scatter(out_v, [orow, ocol + D + cc], rvv)
            return 0

        lax.fori_loop(0, G16, extract, 0)

    pltpu.sync_copy(out_v, out_hbm.at[wid])


@jax.jit
def _meta_gather(nodes3, rels3, node_maps, ent4, relf):
    kern = functools.partial(
        pl.kernel,
        out_type=jax.ShapeDtypeStruct((NW, BPW // 2, 4 * D), jnp.float32),
        mesh=plsc.VectorSubcoreMesh(core_axis_name="c", subcore_axis_name="s"),
        compiler_params=pltpu.CompilerParams(needs_layout_passes=False),
        scratch_types=[
            pltpu.VMEM((CH, CB), jnp.int32),
            pltpu.VMEM((CH, CB), jnp.int32),
            pltpu.VMEM((CH, CB), jnp.int32),
            pltpu.VMEM((CH, CB), jnp.int32),
            pltpu.VMEM((NR * D,), jnp.float32),
            pltpu.VMEM((CH, CB, 4 * D), jnp.float32),
            pltpu.VMEM((BPW // 2, 4 * D), jnp.float32),
            pltpu.SemaphoreType.DMA,
            pltpu.SemaphoreType.DMA,
        ],
    )(_sc_body)
    return kern(nodes3, rels3, node_maps, ent4, relf)


def kernel(nodes, rel_types, node_maps, ent_table, rel_table):
    nodes3 = nodes.astype(jnp.int32).reshape(NW, CH, CB)
    rels3 = rel_types.astype(jnp.int32).reshape(NW, CH, CB)
    ent4 = ent_table.reshape(-1, 4 * D)
    relf = rel_table.reshape(-1)
    out = _meta_gather(nodes3, rels3, node_maps.astype(jnp.int32), ent4, relf)
    return out.reshape(B, 2 * D)


# speedup vs baseline: 1.0511x; 1.0511x over previous
import functools

import jax
import jax.numpy as jnp
from jax import lax
from jax.experimental import pallas as pl
from jax.experimental.pallas import tpu as pltpu
from jax.experimental.pallas import tpu_sc as plsc

B = 16384
D = 32
NR = 200
NC = 2
NS = 16
NW = NC * NS
BPW = B // NW
CB = 128
CH = BPW // CB
G16 = CB // 16


def _sc_body(nodes_hbm, rels_hbm, maps_hbm, ent4_hbm, relf_hbm, out_hbm,
             nodes_v, rels_v, typed_v, gid_v, relf_v, egrp_v, out_v, s1, s2):
    c = lax.axis_index("c")
    s = lax.axis_index("s")
    wid = s * NC + c
    lanes = lax.iota(jnp.int32, 16)

    pltpu.sync_copy(nodes_hbm.at[wid], nodes_v)
    l1 = [pltpu.async_copy(maps_hbm.at[nodes_v.at[j]], typed_v.at[j], s1)
          for j in range(CH)]
    pltpu.sync_copy(rels_hbm.at[wid], rels_v)
    pltpu.sync_copy(relf_hbm, relf_v)

    l2 = []
    for j in range(CH):
        l1[j].wait()

        def gid_row(g, _, j=j):
            sl = pl.ds(g * 16, 16)
            gid_v.at[j][sl] = lax.shift_right_logical(typed_v.at[j][sl], 2)
            return 0

        lax.fori_loop(0, G16, gid_row, 0)
        l2.append(pltpu.async_copy(ent4_hbm.at[gid_v.at[j]], egrp_v.at[j], s2))

    for j in range(CH):
        l2[j].wait()

        def extract(g, _, j=j):
            sl = pl.ds(g * 16, 16)
            tv = typed_v.at[j][sl]
            rv = rels_v.at[j][sl]
            esub = (tv & 3) * D
            rbase = rv * D
            erow = g * 16 + lanes
            for cc in range(D):
                out_v.at[cc][sl] = plsc.load_gather(
                    egrp_v.at[j], [erow, esub + cc])
                out_v.at[D + cc][sl] = plsc.load_gather(
                    relf_v, [rbase + cc])
            return 0

        lax.fori_loop(0, G16, extract, 0)
        col = wid * BPW + j * CB
        pltpu.sync_copy(out_v, out_hbm.at[:, pl.ds(col, CB)])


@jax.jit
def _meta_gather(nodes3, rels3, node_maps, ent4, relf):
    kern = functools.partial(
        pl.kernel,
        out_type=jax.ShapeDtypeStruct((2 * D, B), jnp.float32),
        mesh=plsc.VectorSubcoreMesh(core_axis_name="c", subcore_axis_name="s"),
        compiler_params=pltpu.CompilerParams(needs_layout_passes=False),
        scratch_types=[
            pltpu.VMEM((CH, CB), jnp.int32),
            pltpu.VMEM((CH, CB), jnp.int32),
            pltpu.VMEM((CH, CB), jnp.int32),
            pltpu.VMEM((CH, CB), jnp.int32),
            pltpu.VMEM((NR * D,), jnp.float32),
            pltpu.VMEM((CH, CB, 4 * D), jnp.float32),
            pltpu.VMEM((2 * D, CB), jnp.float32),
            pltpu.SemaphoreType.DMA,
            pltpu.SemaphoreType.DMA,
        ],
    )(_sc_body)
    return kern(nodes3, rels3, node_maps, ent4, relf)


def kernel(nodes, rel_types, node_maps, ent_table, rel_table):
    nodes3 = nodes.astype(jnp.int32).reshape(NW, CH, CB)
    rels3 = rel_types.astype(jnp.int32).reshape(NW, CH, CB)
    ent4 = ent_table.reshape(-1, 4 * D)
    relf = rel_table.reshape(-1)
    outt = _meta_gather(nodes3, rels3, node_maps.astype(jnp.int32), ent4, relf)
    return outt.T


# speedup vs baseline: 1.0737x; 1.0215x over previous
import functools

import jax
import jax.numpy as jnp
from jax import lax
from jax.experimental import pallas as pl
from jax.experimental.pallas import tpu as pltpu
from jax.experimental.pallas import tpu_sc as plsc

B = 16384
D = 32
NR = 200
NC = 2
NS = 16
NW = NC * NS
BPW = B // NW
CB = 128
CH = BPW // CB
G16 = CB // 16


def _sc_body(nodes_hbm, rels_hbm, maps_hbm, entp_hbm, relf_hbm, out_hbm,
             nodes_v, rels_v, typed_v, relf_v, egrp_v, out_v, s1, s2):
    c = lax.axis_index("c")
    s = lax.axis_index("s")
    wid = s * NC + c
    lanes = lax.iota(jnp.int32, 16)

    pltpu.sync_copy(nodes_hbm.at[wid], nodes_v)
    l1 = [pltpu.async_copy(maps_hbm.at[nodes_v.at[j]], typed_v.at[j], s1)
          for j in range(CH)]
    pltpu.sync_copy(rels_hbm.at[wid], rels_v)
    pltpu.sync_copy(relf_hbm, relf_v)

    l2 = [None] * CH

    def fire(j):
        l1[j].wait()

        l2[j] = pltpu.async_copy(entp_hbm.at[typed_v.at[j]], egrp_v.at[j], s2)

    fire(0)
    fire(1)
    fire(2)
    fire(3)

    for j in range(CH):
        l2[j].wait()

        def extract(g, _, j=j):
            sl = pl.ds(g * 16, 16)
            rv = rels_v.at[j][sl]
            rbase = rv * D
            erow = g * 16 + lanes
            for cc in range(D):
                ccv = jnp.full((16,), cc, jnp.int32)
                out_v.at[cc][sl] = plsc.load_gather(
                    egrp_v.at[j], [erow, ccv])
                out_v.at[D + cc][sl] = plsc.load_gather(
                    relf_v, [rbase + cc])
            return 0

        lax.fori_loop(0, G16, extract, 0)
        col = wid * BPW + j * CB
        pltpu.sync_copy(out_v, out_hbm.at[:, pl.ds(col, CB)])


@jax.jit
def _meta_gather(nodes3, rels3, node_maps, ent4, relf):
    kern = functools.partial(
        pl.kernel,
        out_type=jax.ShapeDtypeStruct((2 * D, B), jnp.float32),
        mesh=plsc.VectorSubcoreMesh(core_axis_name="c", subcore_axis_name="s"),
        compiler_params=pltpu.CompilerParams(needs_layout_passes=False),
        scratch_types=[
            pltpu.VMEM((CH, CB), jnp.int32),
            pltpu.VMEM((CH, CB), jnp.int32),
            pltpu.VMEM((CH, CB), jnp.int32),
            pltpu.VMEM((NR * D,), jnp.float32),
            pltpu.VMEM((CH, CB, 128), jnp.float32),
            pltpu.VMEM((2 * D, CB), jnp.float32),
            pltpu.SemaphoreType.DMA,
            pltpu.SemaphoreType.DMA,
        ],
    )(_sc_body)
    return kern(nodes3, rels3, node_maps, ent4, relf)


def kernel(nodes, rel_types, node_maps, ent_table, rel_table):
    nodes3 = nodes.astype(jnp.int32).reshape(NW, CH, CB)
    rels3 = rel_types.astype(jnp.int32).reshape(NW, CH, CB)
    entp = jnp.pad(ent_table, ((0, 0), (0, 128 - D)))
    relf = rel_table.reshape(-1)
    outt = _meta_gather(nodes3, rels3, node_maps.astype(jnp.int32), entp, relf)
    return outt.T
